# baseline (speedup 1.0000x reference)
import functools

import jax
import jax.numpy as jnp
from jax import lax
from jax.experimental import pallas as pl
from jax.experimental.pallas import tpu as pltpu
from jax.experimental.pallas import tpu_sc as plsc

_B = 16384
_D = 32
_PACK = 4
_LINE = _D * _PACK
_NC, _NS = 2, 16
_NW = _NC * _NS
_BPW = _B // _NW
_CHUNK = 64
_NCHUNK = _BPW // _CHUNK
_NBUF = 2

_BLK = 2048

_V = 1000000
_RPW = 4096
_NJ = 62
_S = _RPW * _NJ


def _repack_body(*refs):
    in_refs, out_refs = refs[:16], refs[16:]
    for t in range(4):
        x = jnp.concatenate([in_refs[4 * t + s][...] for s in range(4)],
                            axis=0)
        out_refs[t][...] = x.T


def _tc_repack4(tT0, tT1, tT2, tT3):
    last_blk = (_V + _RPW - 1) // _RPW - 1

    def in_spec(s):
        return pl.BlockSpec(
            (_D, _RPW),
            lambda j, s=s: (0, jnp.minimum(_NJ * s + j, last_blk)))

    out4 = jax.ShapeDtypeStruct((_S, _LINE), jnp.float32)
    return pl.pallas_call(
        _repack_body,
        grid=(_NJ,),
        in_specs=[in_spec(s) for _ in range(4) for s in range(4)],
        out_specs=[pl.BlockSpec((_RPW, _LINE), lambda j: (j, 0))] * 4,
        out_shape=[out4, out4, out4, out4],
        compiler_params=pltpu.CompilerParams(
            dimension_semantics=("parallel",)),
    )(*[t for t in (tT0, tT1, tT2, tT3) for _ in range(4)])


def _sc_gather4(u_idx3, i_idx3, t_ug, t_ig, t_um, t_im):
    mesh = plsc.VectorSubcoreMesh(core_axis_name="c", subcore_axis_name="s")
    out4 = jax.ShapeDtypeStruct((_NW, _NCHUNK, _CHUNK, _LINE), jnp.float32)

    @functools.partial(
        pl.kernel,
        mesh=mesh,
        out_type=[out4, out4, out4, out4],
        compiler_params=pltpu.CompilerParams(use_tc_tiling_on_sc=True),
        scratch_types=[
            pltpu.VMEM((_NCHUNK, _CHUNK), jnp.int32),
            pltpu.VMEM((_NCHUNK, _CHUNK), jnp.int32),
            pltpu.VMEM((_NBUF, _CHUNK, _LINE), jnp.float32),
            pltpu.VMEM((_NBUF, _CHUNK, _LINE), jnp.float32),
            pltpu.VMEM((_NBUF, _CHUNK, _LINE), jnp.float32),
            pltpu.VMEM((_NBUF, _CHUNK, _LINE), jnp.float32),
            pltpu.SemaphoreType.DMA,
        ],
    )
    def k(uidx_hbm, iidx_hbm, ug_hbm, ig_hbm, um_hbm, im_hbm,
          o_ug, o_ig, o_um, o_im,
          uix_v, iix_v, r_ug, r_ig, r_um, r_im, sem):
        wid = lax.axis_index("s") * _NC + lax.axis_index("c")
        pltpu.sync_copy(uidx_hbm.at[wid], uix_v)
        pltpu.sync_copy(iidx_hbm.at[wid], iix_v)
        bufs = (r_ug, r_ig, r_um, r_im)
        outs = (o_ug, o_ig, o_um, o_im)
        tabs = (ug_hbm, ig_hbm, um_hbm, im_hbm)
        ixs = (uix_v, iix_v, uix_v, iix_v)

        def fire(c):
            b = c % _NBUF
            return [pltpu.async_copy(tabs[t].at[ixs[t].at[c]], bufs[t].at[b], sem)
                    for t in range(4)]

        pending = {c: fire(c) for c in range(_NBUF)}
        for c in range(_NCHUNK):
            for cp in pending.pop(c):
                cp.wait()
            b = c % _NBUF
            for t in range(4):
                pltpu.sync_copy(bufs[t].at[b], outs[t].at[wid, c])
            if c + _NBUF < _NCHUNK:
                pending[c + _NBUF] = fire(c + _NBUF)

    return k(u_idx3, i_idx3, t_ug, t_ig, t_um, t_im)


def _select32(wide, masks):
    acc = masks[0] * wide[:, 0:_D]
    for s in range(1, _PACK):
        acc += masks[s] * wide[:, s * _D:(s + 1) * _D]
    return acc


def _bf16(x):
    return x.astype(jnp.bfloat16)


def _mlp_body(ug_ref, ig_ref, um_ref, im_ref, uidx_ref, iidx_ref,
              w1a_ref, w1b_ref, b1_ref, w2_ref, b2_ref, wpa_ref, wpb_ref,
              bp_ref, o_ref):
    uslot = uidx_ref[...] // _S
    islot = iidx_ref[...] // _S
    umask = [(uslot == s).astype(jnp.float32) for s in range(_PACK)]
    imask = [(islot == s).astype(jnp.float32) for s in range(_PACK)]
    ug = _select32(ug_ref[...], umask)
    ig = _select32(ig_ref[...], imask)
    um = _select32(um_ref[...], umask)
    im = _select32(im_ref[...], imask)
    h1 = jnp.dot(_bf16(um), _bf16(w1a_ref[...]),
                 preferred_element_type=jnp.float32)
    h1 += jnp.dot(_bf16(im), _bf16(w1b_ref[...]),
                  preferred_element_type=jnp.float32)
    h1 = jnp.maximum(h1 + b1_ref[...], 0.0)
    h2 = jnp.dot(_bf16(h1), _bf16(w2_ref[...]),
                 preferred_element_type=jnp.float32)
    h2 = jnp.maximum(h2 + b2_ref[...], 0.0)
    g = ug * ig
    p = (jnp.sum(g * wpa_ref[...], axis=1, keepdims=True)
         + jnp.sum(h2 * wpb_ref[...], axis=1, keepdims=True)
         + bp_ref[...])
    o_ref[...] = jax.nn.sigmoid(p)


def _tc_mlp(ug, ig, um, im, uidx, iidx, w1a, w1b, b1r, w2, b2r, wpa, wpb, bpr):
    wide_spec = pl.BlockSpec((_BLK, _LINE), lambda i: (i, 0))
    sel_spec = pl.BlockSpec((_BLK, 1), lambda i: (i, 0))

    def full(shape):
        return pl.BlockSpec(shape, lambda i: (0, 0))

    return pl.pallas_call(
        _mlp_body,
        grid=(_B // _BLK,),
        in_specs=[
            wide_spec, wide_spec, wide_spec, wide_spec,
            sel_spec, sel_spec,
            full((_D, 32)), full((_D, 32)), full((1, 32)),
            full((32, 16)), full((1, 16)),
            full((1, _D)), full((1, 16)), full((1, 1)),
        ],
        out_specs=pl.BlockSpec((_BLK, 1), lambda i: (i, 0)),
        out_shape=jax.ShapeDtypeStruct((_B, 1), jnp.float32),
        compiler_params=pltpu.CompilerParams(
            dimension_semantics=("parallel",)),
    )(ug, ig, um, im, uidx, iidx, w1a, w1b, b1r, w2, b2r, wpa, wpb, bpr)


def kernel(user_indices, item_indices, embed_user_GMF, embed_item_GMF,
           embed_user_MLP, embed_item_MLP, W1, b1, W2, b2, Wp, bp):
    ui = user_indices.astype(jnp.int32)
    ii = item_indices.astype(jnp.int32)
    u3 = (ui % _S).reshape(_NW, _NCHUNK, _CHUNK)
    i3 = (ii % _S).reshape(_NW, _NCHUNK, _CHUNK)
    tables = _tc_repack4(embed_user_GMF.T, embed_item_GMF.T,
                         embed_user_MLP.T, embed_item_MLP.T)
    ug, ig, um, im = _sc_gather4(u3, i3, *tables)
    ug = ug.reshape(_B, _LINE)
    ig = ig.reshape(_B, _LINE)
    um = um.reshape(_B, _LINE)
    im = im.reshape(_B, _LINE)
    w1a, w1b = W1[:_D], W1[_D:]
    wpa = Wp[:_D, 0].reshape(1, _D)
    wpb = Wp[_D:, 0].reshape(1, 16)
    out = _tc_mlp(ug, ig, um, im, ui.reshape(_B, 1), ii.reshape(_B, 1),
                  w1a, w1b, b1.reshape(1, 32),
                  W2, b2.reshape(1, 16), wpa, wpb, bp.reshape(1, 1))
    return out.reshape(-1)


# speedup vs baseline: 1.1005x; 1.1005x over previous
import functools

import jax
import jax.numpy as jnp
from jax import lax
from jax.experimental import pallas as pl
from jax.experimental.pallas import tpu as pltpu
from jax.experimental.pallas import tpu_sc as plsc

_B = 16384
_D = 32
_PACK = 4
_LINE = _D * _PACK
_NC, _NS = 2, 16
_NW = _NC * _NS
_BPW = _B // _NW
_CHUNK = 64
_NCHUNK = _BPW // _CHUNK
_NBUF = 2

_BLK = 2048

_V = 1000000
_RPW = 4096
_NJ = 62
_S = _RPW * _NJ


def _repack_body(*refs):
    in_refs, out_refs = refs[:16], refs[16:]
    for t in range(4):
        x = jnp.concatenate([in_refs[4 * t + s][...] for s in range(4)],
                            axis=0)
        out_refs[t][...] = x.T


def _tc_repack4(tT0, tT1, tT2, tT3):
    last_blk = (_V + _RPW - 1) // _RPW - 1

    def in_spec(s):
        return pl.BlockSpec(
            (_D, _RPW),
            lambda j, s=s: (0, jnp.minimum(_NJ * s + j, last_blk)))

    out4 = jax.ShapeDtypeStruct((_S, _LINE), jnp.float32)
    return pl.pallas_call(
        _repack_body,
        grid=(_NJ,),
        in_specs=[in_spec(s) for _ in range(4) for s in range(4)],
        out_specs=[pl.BlockSpec((_RPW, _LINE), lambda j: (j, 0))] * 4,
        out_shape=[out4, out4, out4, out4],
        compiler_params=pltpu.CompilerParams(
            dimension_semantics=("parallel",)),
    )(*[t for t in (tT0, tT1, tT2, tT3) for _ in range(4)])


def _sc_gather4(u_idx3, i_idx3, t_ug, t_ig, t_um, t_im):
    mesh = plsc.VectorSubcoreMesh(core_axis_name="c", subcore_axis_name="s")
    out4 = jax.ShapeDtypeStruct((_NW, _NCHUNK, _CHUNK, _LINE), jnp.float32)

    @functools.partial(
        pl.kernel,
        mesh=mesh,
        out_type=[out4, out4, out4, out4],
        compiler_params=pltpu.CompilerParams(use_tc_tiling_on_sc=True),
        scratch_types=[
            pltpu.VMEM((_NCHUNK, _CHUNK), jnp.int32),
            pltpu.VMEM((_NCHUNK, _CHUNK), jnp.int32),
            pltpu.VMEM((_NBUF, _CHUNK, _LINE), jnp.float32),
            pltpu.VMEM((_NBUF, _CHUNK, _LINE), jnp.float32),
            pltpu.VMEM((_NBUF, _CHUNK, _LINE), jnp.float32),
            pltpu.VMEM((_NBUF, _CHUNK, _LINE), jnp.float32),
            pltpu.SemaphoreType.DMA,
        ],
    )
    def k(uidx_hbm, iidx_hbm, ug_hbm, ig_hbm, um_hbm, im_hbm,
          o_ug, o_ig, o_um, o_im,
          uix_v, iix_v, r_ug, r_ig, r_um, r_im, sem):
        wid = lax.axis_index("s") * _NC + lax.axis_index("c")
        pltpu.sync_copy(uidx_hbm.at[wid], uix_v)
        pltpu.sync_copy(iidx_hbm.at[wid], iix_v)
        bufs = (r_ug, r_ig, r_um, r_im)
        outs = (o_ug, o_ig, o_um, o_im)
        tabs = (ug_hbm, ig_hbm, um_hbm, im_hbm)
        ixs = (uix_v, iix_v, uix_v, iix_v)

        def fire(c):
            b = c % _NBUF
            return [pltpu.async_copy(tabs[t].at[ixs[t].at[c]], bufs[t].at[b], sem)
                    for t in range(4)]

        pending = {c: fire(c) for c in range(_NBUF)}
        for c in range(_NCHUNK):
            for cp in pending.pop(c):
                cp.wait()
            b = c % _NBUF
            for t in range(4):
                pltpu.sync_copy(bufs[t].at[b], outs[t].at[wid, c])
            if c + _NBUF < _NCHUNK:
                pending[c + _NBUF] = fire(c + _NBUF)

    return k(u_idx3, i_idx3, t_ug, t_ig, t_um, t_im)


def _bf16(x):
    return x.astype(jnp.bfloat16)


def _mlp_body(ug_ref, ig_ref, um_ref, im_ref, umask_ref, imask_ref,
              w1A_ref, w1B_ref, b1_ref, w2_ref, b2_ref, c4w_ref, c4_ref,
              ones_ref, wpb_ref, bp_ref, o_ref):
    umask = umask_ref[...]
    imask = imask_ref[...]
    aug = _bf16(ug_ref[...]) * umask
    aig = _bf16(ig_ref[...]) * imask
    aum = _bf16(um_ref[...]) * umask
    aim = _bf16(im_ref[...]) * imask
    f32 = jnp.float32
    ugw = jnp.dot(aug, c4w_ref[...], preferred_element_type=f32)
    igs = jnp.dot(aig, c4_ref[...], preferred_element_type=f32)
    h1 = jnp.dot(aum, w1A_ref[...], preferred_element_type=f32)
    h1 += jnp.dot(aim, w1B_ref[...], preferred_element_type=f32)
    h1 = jnp.maximum(h1 + b1_ref[...], 0.0)
    h2 = jnp.dot(_bf16(h1), w2_ref[...], preferred_element_type=f32)
    h2 = jnp.maximum(h2 + b2_ref[...], 0.0)
    p = (jnp.dot(_bf16(ugw * igs), ones_ref[...], preferred_element_type=f32)
         + jnp.dot(_bf16(h2), wpb_ref[...], preferred_element_type=f32)
         + bp_ref[...])
    o_ref[...] = jax.nn.sigmoid(p)


def _tc_mlp(ug, ig, um, im, umask, imask, w1A, w1B, b1r, w2, b2r,
            c4w, c4, ones32, wpbc, bpr):
    wide_spec = pl.BlockSpec((_BLK, _LINE), lambda i: (i, 0))

    def full(shape):
        return pl.BlockSpec(shape, lambda i: (0, 0))

    return pl.pallas_call(
        _mlp_body,
        grid=(_B // _BLK,),
        in_specs=[
            wide_spec, wide_spec, wide_spec, wide_spec,
            wide_spec, wide_spec,
            full((_LINE, 32)), full((_LINE, 32)), full((1, 32)),
            full((32, 16)), full((1, 16)),
            full((_LINE, 32)), full((_LINE, 32)),
            full((32, 1)), full((16, 1)), full((1, 1)),
        ],
        out_specs=pl.BlockSpec((_BLK, 1), lambda i: (i, 0)),
        out_shape=jax.ShapeDtypeStruct((_B, 1), jnp.float32),
        compiler_params=pltpu.CompilerParams(
            dimension_semantics=("parallel",)),
    )(ug, ig, um, im, umask, imask, w1A, w1B, b1r, w2, b2r,
      c4w, c4, ones32, wpbc, bpr)


def kernel(user_indices, item_indices, embed_user_GMF, embed_item_GMF,
           embed_user_MLP, embed_item_MLP, W1, b1, W2, b2, Wp, bp):
    ui = user_indices.astype(jnp.int32)
    ii = item_indices.astype(jnp.int32)
    u3 = (ui % _S).reshape(_NW, _NCHUNK, _CHUNK)
    i3 = (ii % _S).reshape(_NW, _NCHUNK, _CHUNK)
    tables = _tc_repack4(embed_user_GMF.T, embed_item_GMF.T,
                         embed_user_MLP.T, embed_item_MLP.T)
    ug, ig, um, im = _sc_gather4(u3, i3, *tables)
    ug = ug.reshape(_B, _LINE)
    ig = ig.reshape(_B, _LINE)
    um = um.reshape(_B, _LINE)
    im = im.reshape(_B, _LINE)
    lane_slot = jnp.arange(_LINE, dtype=jnp.int32) // _D
    umask = _bf16(lane_slot[None, :] == (ui // _S)[:, None])
    imask = _bf16(lane_slot[None, :] == (ii // _S)[:, None])
    w1a, w1b = W1[:_D], W1[_D:]
    w1A = _bf16(jnp.tile(w1a, (_PACK, 1)))
    w1B = _bf16(jnp.tile(w1b, (_PACK, 1)))
    eye32 = jnp.eye(_D, dtype=jnp.float32)
    c4 = _bf16(jnp.tile(eye32, (_PACK, 1)))
    c4w = _bf16(jnp.tile(eye32 * Wp[:_D, 0][None, :], (_PACK, 1)))
    ones32 = _bf16(jnp.ones((_D, 1), jnp.float32))
    wpbc = _bf16(Wp[_D:])
    out = _tc_mlp(ug, ig, um, im, umask, imask,
                  w1A, w1B, b1.reshape(1, 32),
                  _bf16(W2), b2.reshape(1, 16),
                  c4w, c4, ones32, wpbc, bp.reshape(1, 1))
    return out.reshape(-1)


# speedup vs baseline: 1.3176x; 1.1973x over previous
import functools

import jax
import jax.numpy as jnp
from jax import lax
from jax.experimental import pallas as pl
from jax.experimental.pallas import tpu as pltpu
from jax.experimental.pallas import tpu_sc as plsc

_B = 16384
_D = 32
_PACK = 4
_LINE = _D * _PACK
_NC, _NS = 2, 16
_NW = _NC * _NS
_BPW = _B // _NW
_CHUNK = 64
_NCHUNK = _BPW // _CHUNK
_NBUF = 2

_BLK = 2048

_V = 1000000
_RPW = 4096
_NJ = 62
_S = _RPW * _NJ


_RPO = _RPW // 2
_HB = _S // 2 // _RPO


def _repack_body(*refs):
    in_refs, out_refs = refs[:32], refs[32:]
    himask = jnp.uint32(0xFFFF0000)
    for t in range(4):
        lo = jnp.concatenate(
            [in_refs[8 * t + 2 * s][...] for s in range(4)], axis=0)
        hi = jnp.concatenate(
            [in_refs[8 * t + 2 * s + 1][...] for s in range(4)], axis=0)
        blo = jax.lax.bitcast_convert_type(lo.T, jnp.uint32)
        bhi = jax.lax.bitcast_convert_type(hi.T, jnp.uint32)
        packed = (blo >> 16) | (bhi & himask)
        out_refs[t][...] = jax.lax.bitcast_convert_type(packed, jnp.float32)


def _tc_repack4(tT0, tT1, tT2, tT3):
    last_blk = (_V + _RPO - 1) // _RPO - 1

    def in_spec(s, h):
        return pl.BlockSpec(
            (_D, _RPO),
            lambda j, s=s, h=h: (
                0, jnp.minimum(2 * _HB * s + _HB * h + j, last_blk)))

    out4 = jax.ShapeDtypeStruct((_S // 2, _LINE), jnp.float32)
    return pl.pallas_call(
        _repack_body,
        grid=(_HB,),
        in_specs=[in_spec(s, h)
                  for _ in range(4) for s in range(4) for h in range(2)],
        out_specs=[pl.BlockSpec((_RPO, _LINE), lambda j: (j, 0))] * 4,
        out_shape=[out4, out4, out4, out4],
        compiler_params=pltpu.CompilerParams(
            dimension_semantics=("parallel",)),
    )(*[t for t in (tT0, tT1, tT2, tT3) for _ in range(8)])


def _sc_gather4(u_idx3, i_idx3, t_ug, t_ig, t_um, t_im):
    mesh = plsc.VectorSubcoreMesh(core_axis_name="c", subcore_axis_name="s")
    out4 = jax.ShapeDtypeStruct((_NW, _NCHUNK, _CHUNK, _LINE), jnp.float32)

    @functools.partial(
        pl.kernel,
        mesh=mesh,
        out_type=[out4, out4, out4, out4],
        compiler_params=pltpu.CompilerParams(use_tc_tiling_on_sc=True),
        scratch_types=[
            pltpu.VMEM((_NCHUNK, _CHUNK), jnp.int32),
            pltpu.VMEM((_NCHUNK, _CHUNK), jnp.int32),
            pltpu.VMEM((_NBUF, _CHUNK, _LINE), jnp.float32),
            pltpu.VMEM((_NBUF, _CHUNK, _LINE), jnp.float32),
            pltpu.VMEM((_NBUF, _CHUNK, _LINE), jnp.float32),
            pltpu.VMEM((_NBUF, _CHUNK, _LINE), jnp.float32),
            pltpu.SemaphoreType.DMA,
        ],
    )
    def k(uidx_hbm, iidx_hbm, ug_hbm, ig_hbm, um_hbm, im_hbm,
          o_ug, o_ig, o_um, o_im,
          uix_v, iix_v, r_ug, r_ig, r_um, r_im, sem):
        wid = lax.axis_index("s") * _NC + lax.axis_index("c")
        pltpu.sync_copy(uidx_hbm.at[wid], uix_v)
        pltpu.sync_copy(iidx_hbm.at[wid], iix_v)
        bufs = (r_ug, r_ig, r_um, r_im)
        outs = (o_ug, o_ig, o_um, o_im)
        tabs = (ug_hbm, ig_hbm, um_hbm, im_hbm)
        ixs = (uix_v, iix_v, uix_v, iix_v)

        def fire(c):
            b = c % _NBUF
            return [pltpu.async_copy(tabs[t].at[ixs[t].at[c]], bufs[t].at[b], sem)
                    for t in range(4)]

        pending = {c: fire(c) for c in range(_NBUF)}
        for c in range(_NCHUNK):
            for cp in pending.pop(c):
                cp.wait()
            b = c % _NBUF
            for t in range(4):
                pltpu.sync_copy(bufs[t].at[b], outs[t].at[wid, c])
            if c + _NBUF < _NCHUNK:
                pending[c + _NBUF] = fire(c + _NBUF)

    return k(u_idx3, i_idx3, t_ug, t_ig, t_um, t_im)


def _bf16(x):
    return x.astype(jnp.bfloat16)


def _unpack_sel(w_f32, m0, m1):
    bits = jax.lax.bitcast_convert_type(w_f32, jnp.uint32)
    even = jax.lax.bitcast_convert_type(bits << 16, jnp.float32)
    odd = jax.lax.bitcast_convert_type(bits & jnp.uint32(0xFFFF0000), jnp.float32)
    return _bf16(even) * m0 + _bf16(odd) * m1


def _mlp_body(ug_ref, ig_ref, um_ref, im_ref,
              um0_ref, um1_ref, im0_ref, im1_ref,
              w1A_ref, w1B_ref, b1_ref, w2_ref, b2_ref, c4w_ref, c4_ref,
              ones_ref, wpb_ref, bp_ref, o_ref):
    um0, um1 = um0_ref[...], um1_ref[...]
    im0, im1 = im0_ref[...], im1_ref[...]
    aug = _unpack_sel(ug_ref[...], um0, um1)
    aig = _unpack_sel(ig_ref[...], im0, im1)
    aum = _unpack_sel(um_ref[...], um0, um1)
    aim = _unpack_sel(im_ref[...], im0, im1)
    f32 = jnp.float32
    ugw = jnp.dot(aug, c4w_ref[...], preferred_element_type=f32)
    igs = jnp.dot(aig, c4_ref[...], preferred_element_type=f32)
    h1 = jnp.dot(aum, w1A_ref[...], preferred_element_type=f32)
    h1 += jnp.dot(aim, w1B_ref[...], preferred_element_type=f32)
    h1 = jnp.maximum(h1 + b1_ref[...], 0.0)
    h2 = jnp.dot(_bf16(h1), w2_ref[...], preferred_element_type=f32)
    h2 = jnp.maximum(h2 + b2_ref[...], 0.0)
    p = (jnp.dot(_bf16(ugw * igs), ones_ref[...], preferred_element_type=f32)
         + jnp.dot(_bf16(h2), wpb_ref[...], preferred_element_type=f32)
         + bp_ref[...])
    o_ref[...] = jax.nn.sigmoid(p)


def _tc_mlp(ug, ig, um, im, um0, um1, im0, im1, w1A, w1B, b1r, w2, b2r,
            c4w, c4, ones32, wpbc, bpr):
    wide_spec = pl.BlockSpec((_BLK, _LINE), lambda i: (i, 0))

    def full(shape):
        return pl.BlockSpec(shape, lambda i: (0, 0))

    return pl.pallas_call(
        _mlp_body,
        grid=(_B // _BLK,),
        in_specs=[
            wide_spec, wide_spec, wide_spec, wide_spec,
            wide_spec, wide_spec, wide_spec, wide_spec,
            full((_LINE, 32)), full((_LINE, 32)), full((1, 32)),
            full((32, 16)), full((1, 16)),
            full((_LINE, 32)), full((_LINE, 32)),
            full((32, 1)), full((16, 1)), full((1, 1)),
        ],
        out_specs=pl.BlockSpec((_BLK, 1), lambda i: (i, 0)),
        out_shape=jax.ShapeDtypeStruct((_B, 1), jnp.float32),
        compiler_params=pltpu.CompilerParams(
            dimension_semantics=("parallel",)),
    )(ug, ig, um, im, um0, um1, im0, im1, w1A, w1B, b1r, w2, b2r,
      c4w, c4, ones32, wpbc, bpr)


def kernel(user_indices, item_indices, embed_user_GMF, embed_item_GMF,
           embed_user_MLP, embed_item_MLP, W1, b1, W2, b2, Wp, bp):
    ui = user_indices.astype(jnp.int32)
    ii = item_indices.astype(jnp.int32)
    uq = ui % _S
    iq = ii % _S
    u3 = (uq % (_S // 2)).reshape(_NW, _NCHUNK, _CHUNK)
    i3 = (iq % (_S // 2)).reshape(_NW, _NCHUNK, _CHUNK)
    tables = _tc_repack4(embed_user_GMF.T, embed_item_GMF.T,
                         embed_user_MLP.T, embed_item_MLP.T)
    ug, ig, um, im = _sc_gather4(u3, i3, *tables)
    ug = ug.reshape(_B, _LINE)
    ig = ig.reshape(_B, _LINE)
    um = um.reshape(_B, _LINE)
    im = im.reshape(_B, _LINE)
    lane_slot = jnp.arange(_LINE, dtype=jnp.int32) // _D
    uhit = lane_slot[None, :] == (ui // _S)[:, None]
    ihit = lane_slot[None, :] == (ii // _S)[:, None]
    um0 = _bf16(uhit & (uq < _S // 2)[:, None])
    um1 = _bf16(uhit & (uq >= _S // 2)[:, None])
    im0 = _bf16(ihit & (iq < _S // 2)[:, None])
    im1 = _bf16(ihit & (iq >= _S // 2)[:, None])
    w1a, w1b = W1[:_D], W1[_D:]
    w1A = _bf16(jnp.tile(w1a, (_PACK, 1)))
    w1B = _bf16(jnp.tile(w1b, (_PACK, 1)))
    eye32 = jnp.eye(_D, dtype=jnp.float32)
    c4 = _bf16(jnp.tile(eye32, (_PACK, 1)))
    c4w = _bf16(jnp.tile(eye32 * Wp[:_D, 0][None, :], (_PACK, 1)))
    ones32 = _bf16(jnp.ones((_D, 1), jnp.float32))
    wpbc = _bf16(Wp[_D:])
    out = _tc_mlp(ug, ig, um, im, um0, um1, im0, im1,
                  w1A, w1B, b1.reshape(1, 32),
                  _bf16(W2), b2.reshape(1, 16),
                  c4w, c4, ones32, wpbc, bp.reshape(1, 1))
    return out.reshape(-1)


# speedup vs baseline: 1.3261x; 1.0064x over previous
import functools

import jax
import jax.numpy as jnp
from jax import lax
from jax.experimental import pallas as pl
from jax.experimental.pallas import tpu as pltpu
from jax.experimental.pallas import tpu_sc as plsc

_B = 16384
_D = 32
_PACK = 4
_LINE = _D * _PACK
_NC, _NS = 2, 16
_NW = _NC * _NS
_BPW = _B // _NW
_CHUNK = 64
_NCHUNK = _BPW // _CHUNK
_NBUF = 2

_BLK = 2048

_V = 1000000
_RPW = 4096
_NJ = 62
_S = _RPW * _NJ


_RPO = 4096
_HB = _S // 2 // _RPO


def _repack_body(*refs):
    in_refs, out_refs = refs[:32], refs[32:]
    himask = jnp.uint32(0xFFFF0000)
    for t in range(4):
        lo = jnp.concatenate(
            [in_refs[8 * t + 2 * s][...] for s in range(4)], axis=0)
        hi = jnp.concatenate(
            [in_refs[8 * t + 2 * s + 1][...] for s in range(4)], axis=0)
        blo = jax.lax.bitcast_convert_type(lo.T, jnp.uint32)
        bhi = jax.lax.bitcast_convert_type(hi.T, jnp.uint32)
        packed = (blo >> 16) | (bhi & himask)
        out_refs[t][...] = jax.lax.bitcast_convert_type(packed, jnp.float32)


def _tc_repack4(tT0, tT1, tT2, tT3):
    last_blk = (_V + _RPO - 1) // _RPO - 1

    def in_spec(s, h):
        return pl.BlockSpec(
            (_D, _RPO),
            lambda j, s=s, h=h: (
                0, jnp.minimum(2 * _HB * s + _HB * h + j, last_blk)))

    out4 = jax.ShapeDtypeStruct((_S // 2, _LINE), jnp.float32)
    return pl.pallas_call(
        _repack_body,
        grid=(_HB,),
        in_specs=[in_spec(s, h)
                  for _ in range(4) for s in range(4) for h in range(2)],
        out_specs=[pl.BlockSpec((_RPO, _LINE), lambda j: (j, 0))] * 4,
        out_shape=[out4, out4, out4, out4],
        compiler_params=pltpu.CompilerParams(
            dimension_semantics=("parallel",)),
    )(*[t for t in (tT0, tT1, tT2, tT3) for _ in range(8)])


def _sc_gather4(u_idx3, i_idx3, t_ug, t_ig, t_um, t_im):
    mesh = plsc.VectorSubcoreMesh(core_axis_name="c", subcore_axis_name="s")
    out4 = jax.ShapeDtypeStruct((_NW, _NCHUNK, _CHUNK, _LINE), jnp.float32)

    @functools.partial(
        pl.kernel,
        mesh=mesh,
        out_type=[out4, out4, out4, out4],
        compiler_params=pltpu.CompilerParams(use_tc_tiling_on_sc=True),
        scratch_types=[
            pltpu.VMEM((_NCHUNK, _CHUNK), jnp.int32),
            pltpu.VMEM((_NCHUNK, _CHUNK), jnp.int32),
            pltpu.VMEM((_NBUF, _CHUNK, _LINE), jnp.float32),
            pltpu.VMEM((_NBUF, _CHUNK, _LINE), jnp.float32),
            pltpu.VMEM((_NBUF, _CHUNK, _LINE), jnp.float32),
            pltpu.VMEM((_NBUF, _CHUNK, _LINE), jnp.float32),
            pltpu.SemaphoreType.DMA,
        ],
    )
    def k(uidx_hbm, iidx_hbm, ug_hbm, ig_hbm, um_hbm, im_hbm,
          o_ug, o_ig, o_um, o_im,
          uix_v, iix_v, r_ug, r_ig, r_um, r_im, sem):
        wid = lax.axis_index("s") * _NC + lax.axis_index("c")
        pltpu.sync_copy(uidx_hbm.at[wid], uix_v)
        pltpu.sync_copy(iidx_hbm.at[wid], iix_v)
        bufs = (r_ug, r_ig, r_um, r_im)
        outs = (o_ug, o_ig, o_um, o_im)
        tabs = (ug_hbm, ig_hbm, um_hbm, im_hbm)
        ixs = (uix_v, iix_v, uix_v, iix_v)

        def fire(c):
            b = c % _NBUF
            return [pltpu.async_copy(tabs[t].at[ixs[t].at[c]], bufs[t].at[b], sem)
                    for t in range(4)]

        pending = {c: fire(c) for c in range(_NBUF)}
        for c in range(_NCHUNK):
            for cp in pending.pop(c):
                cp.wait()
            b = c % _NBUF
            for t in range(4):
                pltpu.sync_copy(bufs[t].at[b], outs[t].at[wid, c])
            if c + _NBUF < _NCHUNK:
                pending[c + _NBUF] = fire(c + _NBUF)

    return k(u_idx3, i_idx3, t_ug, t_ig, t_um, t_im)


def _bf16(x):
    return x.astype(jnp.bfloat16)


def _unpack_sel(w_f32, m0, m1):
    bits = jax.lax.bitcast_convert_type(w_f32, jnp.uint32)
    even = jax.lax.bitcast_convert_type(bits << 16, jnp.float32)
    odd = jax.lax.bitcast_convert_type(bits & jnp.uint32(0xFFFF0000), jnp.float32)
    return _bf16(even) * m0 + _bf16(odd) * m1


def _mlp_body(ug_ref, ig_ref, um_ref, im_ref,
              um0_ref, um1_ref, im0_ref, im1_ref,
              w1A_ref, w1B_ref, b1_ref, w2_ref, b2_ref, c4w_ref, c4_ref,
              ones_ref, wpb_ref, bp_ref, o_ref):
    um0, um1 = um0_ref[...], um1_ref[...]
    im0, im1 = im0_ref[...], im1_ref[...]
    aug = _unpack_sel(ug_ref[...], um0, um1)
    aig = _unpack_sel(ig_ref[...], im0, im1)
    aum = _unpack_sel(um_ref[...], um0, um1)
    aim = _unpack_sel(im_ref[...], im0, im1)
    f32 = jnp.float32
    ugw = jnp.dot(aug, c4w_ref[...], preferred_element_type=f32)
    igs = jnp.dot(aig, c4_ref[...], preferred_element_type=f32)
    h1 = jnp.dot(aum, w1A_ref[...], preferred_element_type=f32)
    h1 += jnp.dot(aim, w1B_ref[...], preferred_element_type=f32)
    h1 = jnp.maximum(h1 + b1_ref[...], 0.0)
    h2 = jnp.dot(_bf16(h1), w2_ref[...], preferred_element_type=f32)
    h2 = jnp.maximum(h2 + b2_ref[...], 0.0)
    p = (jnp.dot(_bf16(ugw * igs), ones_ref[...], preferred_element_type=f32)
         + jnp.dot(_bf16(h2), wpb_ref[...], preferred_element_type=f32)
         + bp_ref[...])
    o_ref[...] = jax.nn.sigmoid(p)


def _tc_mlp(ug, ig, um, im, um0, um1, im0, im1, w1A, w1B, b1r, w2, b2r,
            c4w, c4, ones32, wpbc, bpr):
    wide_spec = pl.BlockSpec((_BLK, _LINE), lambda i: (i, 0))

    def full(shape):
        return pl.BlockSpec(shape, lambda i: (0, 0))

    return pl.pallas_call(
        _mlp_body,
        grid=(_B // _BLK,),
        in_specs=[
            wide_spec, wide_spec, wide_spec, wide_spec,
            wide_spec, wide_spec, wide_spec, wide_spec,
            full((_LINE, 32)), full((_LINE, 32)), full((1, 32)),
            full((32, 16)), full((1, 16)),
            full((_LINE, 32)), full((_LINE, 32)),
            full((32, 1)), full((16, 1)), full((1, 1)),
        ],
        out_specs=pl.BlockSpec((_BLK, 1), lambda i: (i, 0)),
        out_shape=jax.ShapeDtypeStruct((_B, 1), jnp.float32),
        compiler_params=pltpu.CompilerParams(
            dimension_semantics=("parallel",)),
    )(ug, ig, um, im, um0, um1, im0, im1, w1A, w1B, b1r, w2, b2r,
      c4w, c4, ones32, wpbc, bpr)


def kernel(user_indices, item_indices, embed_user_GMF, embed_item_GMF,
           embed_user_MLP, embed_item_MLP, W1, b1, W2, b2, Wp, bp):
    ui = user_indices.astype(jnp.int32)
    ii = item_indices.astype(jnp.int32)
    uq = ui % _S
    iq = ii % _S
    u3 = (uq % (_S // 2)).reshape(_NW, _NCHUNK, _CHUNK)
    i3 = (iq % (_S // 2)).reshape(_NW, _NCHUNK, _CHUNK)
    tables = _tc_repack4(embed_user_GMF.T, embed_item_GMF.T,
                         embed_user_MLP.T, embed_item_MLP.T)
    ug, ig, um, im = _sc_gather4(u3, i3, *tables)
    ug = ug.reshape(_B, _LINE)
    ig = ig.reshape(_B, _LINE)
    um = um.reshape(_B, _LINE)
    im = im.reshape(_B, _LINE)
    lane_slot = jnp.arange(_LINE, dtype=jnp.int32) // _D
    uhit = lane_slot[None, :] == (ui // _S)[:, None]
    ihit = lane_slot[None, :] == (ii // _S)[:, None]
    um0 = _bf16(uhit & (uq < _S // 2)[:, None])
    um1 = _bf16(uhit & (uq >= _S // 2)[:, None])
    im0 = _bf16(ihit & (iq < _S // 2)[:, None])
    im1 = _bf16(ihit & (iq >= _S // 2)[:, None])
    w1a, w1b = W1[:_D], W1[_D:]
    w1A = _bf16(jnp.tile(w1a, (_PACK, 1)))
    w1B = _bf16(jnp.tile(w1b, (_PACK, 1)))
    eye32 = jnp.eye(_D, dtype=jnp.float32)
    c4 = _bf16(jnp.tile(eye32, (_PACK, 1)))
    c4w = _bf16(jnp.tile(eye32 * Wp[:_D, 0][None, :], (_PACK, 1)))
    ones32 = _bf16(jnp.ones((_D, 1), jnp.float32))
    wpbc = _bf16(Wp[_D:])
    out = _tc_mlp(ug, ig, um, im, um0, um1, im0, im1,
                  w1A, w1B, b1.reshape(1, 32),
                  _bf16(W2), b2.reshape(1, 16),
                  c4w, c4, ones32, wpbc, bp.reshape(1, 1))
    return out.reshape(-1)
